# baseline (speedup 1.0000x reference)
import functools

import jax
import jax.numpy as jnp
from jax import lax
from jax.experimental import pallas as pl
from jax.experimental.pallas import tpu as pltpu
from jax.experimental.pallas import tpu_sc as plsc

N_NODES = 10000
N_EDGES = 320000
D_FEAT = 128
HIDDEN = 256
OUT_CH = 64

NC = 2
NS = 16
NW = NC * NS
EPW = N_EDGES // NW
CH = 80
NCHUNK = EPW // CH
SPAN = 32
EPAD = 10112
RPT = 1000
ZROWS = 40
DEGC = 1000


def _sc_aggregate(x, ei4):
    mesh = plsc.VectorSubcoreMesh(core_axis_name="c", subcore_axis_name="s")

    @functools.partial(
        pl.kernel,
        out_type=[
            jax.ShapeDtypeStruct((NC, N_NODES, D_FEAT), jnp.float32),
            jax.ShapeDtypeStruct((NC * N_NODES,), jnp.float32),
        ],
        mesh=mesh,
        scratch_types=[
            pltpu.VMEM((SPAN * CH,), jnp.int32),
            pltpu.VMEM((SPAN * CH,), jnp.int32),
            pltpu.VMEM((CH, D_FEAT), jnp.float32),
            pltpu.VMEM((CH, D_FEAT), jnp.float32),
            pltpu.VMEM((CH, D_FEAT), jnp.float32),
            pltpu.VMEM((CH,), jnp.float32),
            pltpu.VMEM((1008,), jnp.float32),
            pltpu.VMEM_SHARED((N_NODES, D_FEAT), jnp.float32),
            pltpu.VMEM_SHARED((N_NODES,), jnp.float32),
            pltpu.SemaphoreType.DMA,
            pltpu.SemaphoreType.DMA,
            pltpu.SemaphoreType.DMA,
            pltpu.SemaphoreType.DMA,
            pltpu.SemaphoreType.DMA,
            pltpu.SemaphoreType.DMA,
            pltpu.SemaphoreType.DMA,
        ],
    )
    def k(x_hbm, ei_hbm, agg_out, deg_out,
          srcv, dstv, rows, rows1, rows2, ones, zdeg, agg_sh, deg_sh,
          gsem0, gsem1, gsem2, ssem0, ssem1, ssem2, osem):
        c = lax.axis_index("c")
        s = lax.axis_index("s")
        wid = c * NS + s

        zeros16 = jnp.zeros((16,), jnp.float32)

        def zero_rows(i, _):
            r = i // (D_FEAT // 16)
            col = (i % (D_FEAT // 16)) * 16
            rows[r, pl.ds(col, 16)] = zeros16
            return 0

        lax.fori_loop(0, CH * (D_FEAT // 16), zero_rows, 0)

        def zero_zdeg(i, _):
            zdeg[pl.ds(i * 16, 16)] = zeros16
            return 0

        lax.fori_loop(0, 1008 // 16, zero_zdeg, 0)

        ones16 = jnp.ones((16,), jnp.float32)

        def fill_ones(i, _):
            ones[pl.ds(i * 16, 16)] = ones16
            return 0

        lax.fori_loop(0, CH // 16, fill_ones, 0)

        @pl.when(s < N_NODES // RPT)
        def _():
            def zero_agg(t, _):
                pltpu.async_copy(rows.at[pl.ds(0, ZROWS), :],
                                 agg_sh.at[pl.ds(s * RPT + t * ZROWS, ZROWS), :],
                                 ssem0)
                return 0

            lax.fori_loop(0, RPT // ZROWS, zero_agg, 0)
            pltpu.sync_copy(zdeg.at[pl.ds(0, DEGC)], deg_sh.at[pl.ds(s * DEGC, DEGC)])

            def drain_zero(t, _):
                pltpu.make_async_copy(rows.at[pl.ds(0, ZROWS), :],
                                      agg_sh.at[pl.ds(s * RPT, ZROWS), :],
                                      ssem0).wait()
                return 0

            lax.fori_loop(0, RPT // ZROWS, drain_zero, 0)

        plsc.subcore_barrier()

        bufs = (rows, rows1, rows2)
        gsems = (gsem0, gsem1, gsem2)
        ssems = (ssem0, ssem1, ssem2)

        def run_span(base, nch):
            ne = -(-nch * CH // 128) * 128
            pltpu.sync_copy(ei_hbm.at[0, wid, pl.ds(base * CH, ne)],
                            srcv.at[pl.ds(0, ne)])
            pltpu.sync_copy(ei_hbm.at[1, wid, pl.ds(base * CH, ne)],
                            dstv.at[pl.ds(0, ne)])

            def sidx(c):
                return srcv.at[pl.ds(c * CH, CH)]

            def didx(c):
                return dstv.at[pl.ds(c * CH, CH)]

            pltpu.async_copy(x_hbm.at[sidx(0)], rows, gsem0)

            def triple(t, _):
                for j in range(3):
                    c = 3 * t + j
                    jn = (j + 1) % 3

                    @pl.when(jnp.logical_and(c >= 2, c < nch))
                    def _(jn=jn):
                        pltpu.make_async_copy(
                            bufs[jn], agg_sh.at[didx(0)], ssems[jn]).wait()

                    @pl.when(c + 1 < nch)
                    def _(jn=jn, c=c):
                        pltpu.async_copy(x_hbm.at[sidx(c + 1)], bufs[jn],
                                         gsems[jn])

                    @pl.when(c < nch)
                    def _(j=j, c=c):
                        pltpu.make_async_copy(x_hbm.at[sidx(0)], bufs[j],
                                              gsems[j]).wait()
                        pltpu.async_copy(ones, deg_sh.at[didx(c)], osem,
                                         add=True)
                        pltpu.async_copy(bufs[j], agg_sh.at[didx(c)],
                                         ssems[j], add=True)

                    @pl.when(jnp.logical_and(c >= 2, c < nch))
                    def _():
                        pltpu.make_async_copy(ones, deg_sh.at[didx(0)],
                                              osem).wait()
                return 0

            lax.fori_loop(0, (nch + 2) // 3, triple, 0)

            for c in (nch - 2, nch - 1):
                pltpu.make_async_copy(bufs[c % 3], agg_sh.at[didx(0)],
                                      ssems[c % 3]).wait()
                pltpu.make_async_copy(ones, deg_sh.at[didx(0)], osem).wait()

        run_span(0, SPAN)
        run_span(SPAN, SPAN)
        run_span(2 * SPAN, SPAN)
        run_span(3 * SPAN, NCHUNK - 3 * SPAN)

        plsc.subcore_barrier()

        @pl.when(s < N_NODES // RPT)
        def _():
            pltpu.async_copy(agg_sh.at[pl.ds(s * RPT, RPT), :],
                             agg_out.at[c, pl.ds(s * RPT, RPT), :], ssem0)
            pltpu.sync_copy(deg_sh.at[pl.ds(s * DEGC, DEGC)], zdeg.at[pl.ds(0, DEGC)])
            pltpu.sync_copy(zdeg.at[pl.ds(0, DEGC)],
                            deg_out.at[pl.ds(c * N_NODES + s * DEGC, DEGC)])
            pltpu.make_async_copy(agg_sh.at[pl.ds(s * RPT, RPT), :],
                                  agg_out.at[c, pl.ds(s * RPT, RPT), :],
                                  ssem0).wait()

    return k(x, ei4)


BN = 1280


def _tc_mlp_body(x_ref, a0_ref, a1_ref, degt_ref, w0a_ref, w0b_ref, b0_ref,
                 w1_ref, b1_ref, out_ref):
    deg = jnp.maximum(degt_ref[:, 0:1] + degt_ref[:, 1:2], 1.0)
    agg = (a0_ref[0] + a1_ref[0]) / deg
    h = (jnp.dot(x_ref[...], w0a_ref[...], preferred_element_type=jnp.float32)
         + jnp.dot(agg, w0b_ref[...], preferred_element_type=jnp.float32)
         + b0_ref[...])
    h = jnp.maximum(h, 0.0)
    o = jnp.dot(h, w1_ref[...], preferred_element_type=jnp.float32) + b1_ref[...]
    n2 = jnp.sum(o * o, axis=1, keepdims=True)
    out_ref[...] = jnp.transpose(o * lax.rsqrt(jnp.maximum(n2, 1e-24)))


def _tc_mlp(x, agg_p, degt, W0, b0, W1, b1):
    grid = (-(-N_NODES // BN),)
    return pl.pallas_call(
        _tc_mlp_body,
        grid=grid,
        in_specs=[
            pl.BlockSpec((BN, D_FEAT), lambda i: (i, 0)),
            pl.BlockSpec((1, BN, D_FEAT), lambda i: (0, i, 0)),
            pl.BlockSpec((1, BN, D_FEAT), lambda i: (1, i, 0)),
            pl.BlockSpec((BN, 2), lambda i: (i, 0)),
            pl.BlockSpec((D_FEAT, HIDDEN), lambda i: (0, 0)),
            pl.BlockSpec((D_FEAT, HIDDEN), lambda i: (1, 0)),
            pl.BlockSpec((1, HIDDEN), lambda i: (0, 0)),
            pl.BlockSpec((HIDDEN, OUT_CH), lambda i: (0, 0)),
            pl.BlockSpec((1, OUT_CH), lambda i: (0, 0)),
        ],
        out_specs=pl.BlockSpec((OUT_CH, BN), lambda i: (0, i)),
        out_shape=jax.ShapeDtypeStruct((OUT_CH, N_NODES), jnp.float32),
    )(x, agg_p, agg_p, degt, W0, W0, b0, W1, b1)


def kernel(x, edge_index, W0, b0, W1, b1):
    ei3 = jnp.pad(edge_index.reshape(2, NW, EPW),
                  ((0, 0), (0, 0), (0, EPAD - EPW)))
    agg_p, deg_p = _sc_aggregate(x, ei3)
    degt = deg_p.reshape(NC, N_NODES).T
    out_t = _tc_mlp(x, agg_p, degt, W0, b0.reshape(1, HIDDEN),
                    W1, b1.reshape(1, OUT_CH))
    return out_t.T


# speedup vs baseline: 1.0152x; 1.0152x over previous
import functools

import jax
import jax.numpy as jnp
from jax import lax
from jax.experimental import pallas as pl
from jax.experimental.pallas import tpu as pltpu
from jax.experimental.pallas import tpu_sc as plsc

N_NODES = 10000
N_EDGES = 320000
D_FEAT = 128
HIDDEN = 256
OUT_CH = 64

NC = 2
NS = 16
NW = NC * NS
EPW = N_EDGES // NW
CH = 80
NCHUNK = EPW // CH
SPAN = 32
EPAD = 10112
RPT = 1000
ZROWS = 40
DEGC = 1000


def _sc_aggregate(x, ei4):
    mesh = plsc.VectorSubcoreMesh(core_axis_name="c", subcore_axis_name="s")

    @functools.partial(
        pl.kernel,
        out_type=[
            jax.ShapeDtypeStruct((NC, N_NODES, D_FEAT), jnp.float32),
            jax.ShapeDtypeStruct((NC * N_NODES,), jnp.float32),
        ],
        mesh=mesh,
        scratch_types=[
            pltpu.VMEM((SPAN * CH,), jnp.int32),
            pltpu.VMEM((SPAN * CH,), jnp.int32),
            pltpu.VMEM((CH, D_FEAT), jnp.float32),
            pltpu.VMEM((CH, D_FEAT), jnp.float32),
            pltpu.VMEM((CH, D_FEAT), jnp.float32),
            pltpu.VMEM((CH,), jnp.float32),
            pltpu.VMEM((1008,), jnp.float32),
            pltpu.VMEM_SHARED((N_NODES, D_FEAT), jnp.float32),
            pltpu.VMEM_SHARED((N_NODES,), jnp.float32),
            pltpu.SemaphoreType.DMA,
            pltpu.SemaphoreType.DMA,
            pltpu.SemaphoreType.DMA,
            pltpu.SemaphoreType.DMA,
            pltpu.SemaphoreType.DMA,
            pltpu.SemaphoreType.DMA,
            pltpu.SemaphoreType.DMA,
        ],
    )
    def k(x_hbm, ei_hbm, agg_out, deg_out,
          srcv, dstv, rows, rows1, rows2, ones, zdeg, agg_sh, deg_sh,
          gsem0, gsem1, gsem2, ssem0, ssem1, ssem2, osem):
        c = lax.axis_index("c")
        s = lax.axis_index("s")
        wid = c * NS + s

        zeros16 = jnp.zeros((16,), jnp.float32)

        def zero_rows(i, _):
            r = i // (D_FEAT // 16)
            col = (i % (D_FEAT // 16)) * 16
            rows[r, pl.ds(col, 16)] = zeros16
            return 0

        lax.fori_loop(0, CH * (D_FEAT // 16), zero_rows, 0)

        def zero_zdeg(i, _):
            zdeg[pl.ds(i * 16, 16)] = zeros16
            return 0

        lax.fori_loop(0, 1008 // 16, zero_zdeg, 0)

        ones16 = jnp.ones((16,), jnp.float32)

        def fill_ones(i, _):
            ones[pl.ds(i * 16, 16)] = ones16
            return 0

        lax.fori_loop(0, CH // 16, fill_ones, 0)

        @pl.when(s < N_NODES // RPT)
        def _():
            def zero_agg(t, _):
                pltpu.async_copy(rows.at[pl.ds(0, ZROWS), :],
                                 agg_sh.at[pl.ds(s * RPT + t * ZROWS, ZROWS), :],
                                 ssem0)
                return 0

            lax.fori_loop(0, RPT // ZROWS, zero_agg, 0)
            pltpu.sync_copy(zdeg.at[pl.ds(0, DEGC)], deg_sh.at[pl.ds(s * DEGC, DEGC)])

            def drain_zero(t, _):
                pltpu.make_async_copy(rows.at[pl.ds(0, ZROWS), :],
                                      agg_sh.at[pl.ds(s * RPT, ZROWS), :],
                                      ssem0).wait()
                return 0

            lax.fori_loop(0, RPT // ZROWS, drain_zero, 0)

        plsc.subcore_barrier()

        bufs = (rows, rows1, rows2)
        gsems = (gsem0, gsem1, gsem2)
        ssems = (ssem0, ssem1, ssem2)

        def run_span(base, nch):
            ne = -(-nch * CH // 128) * 128
            pltpu.sync_copy(ei_hbm.at[0, wid, pl.ds(base * CH, ne)],
                            srcv.at[pl.ds(0, ne)])
            pltpu.sync_copy(ei_hbm.at[1, wid, pl.ds(base * CH, ne)],
                            dstv.at[pl.ds(0, ne)])

            def sidx(c):
                return srcv.at[pl.ds(c * CH, CH)]

            def didx(c):
                return dstv.at[pl.ds(c * CH, CH)]

            pltpu.async_copy(x_hbm.at[sidx(0)], rows, gsem0)

            def triple(t, _):
                for j in range(3):
                    c = 3 * t + j
                    jn = (j + 1) % 3

                    @pl.when(jnp.logical_and(c >= 2, c < nch))
                    def _(jn=jn):
                        pltpu.make_async_copy(
                            bufs[jn], agg_sh.at[didx(0)], ssems[jn]).wait()

                    @pl.when(c + 1 < nch)
                    def _(jn=jn, c=c):
                        pltpu.async_copy(x_hbm.at[sidx(c + 1)], bufs[jn],
                                         gsems[jn])

                    @pl.when(c < nch)
                    def _(j=j, c=c):
                        pltpu.make_async_copy(x_hbm.at[sidx(0)], bufs[j],
                                              gsems[j]).wait()
                        pltpu.async_copy(ones, deg_sh.at[didx(c)], osem,
                                         add=True)
                        pltpu.async_copy(bufs[j], agg_sh.at[didx(c)],
                                         ssems[j], add=True)

                    @pl.when(jnp.logical_and(c >= 2, c < nch))
                    def _():
                        pltpu.make_async_copy(ones, deg_sh.at[didx(0)],
                                              osem).wait()
                return 0

            lax.fori_loop(0, (nch + 2) // 3, triple, 0)

            for c in (nch - 2, nch - 1):
                pltpu.make_async_copy(bufs[c % 3], agg_sh.at[didx(0)],
                                      ssems[c % 3]).wait()
                pltpu.make_async_copy(ones, deg_sh.at[didx(0)], osem).wait()

        run_span(0, SPAN)
        run_span(SPAN, SPAN)
        run_span(2 * SPAN, SPAN)
        run_span(3 * SPAN, NCHUNK - 3 * SPAN)

        plsc.subcore_barrier()

        @pl.when(s < N_NODES // RPT)
        def _():
            pltpu.async_copy(agg_sh.at[pl.ds(s * RPT, RPT), :],
                             agg_out.at[c, pl.ds(s * RPT, RPT), :], ssem0)
            pltpu.sync_copy(deg_sh.at[pl.ds(s * DEGC, DEGC)], zdeg.at[pl.ds(0, DEGC)])
            pltpu.sync_copy(zdeg.at[pl.ds(0, DEGC)],
                            deg_out.at[pl.ds(c * N_NODES + s * DEGC, DEGC)])
            pltpu.make_async_copy(agg_sh.at[pl.ds(s * RPT, RPT), :],
                                  agg_out.at[c, pl.ds(s * RPT, RPT), :],
                                  ssem0).wait()

    return k(x, ei4)


BN = 2560


def _tc_mlp_body(x_ref, a0_ref, a1_ref, degt_ref, w0a_ref, w0b_ref, b0_ref,
                 w1_ref, b1_ref, out_ref):
    deg = jnp.maximum(degt_ref[:, 0:1] + degt_ref[:, 1:2], 1.0)
    agg = (a0_ref[0] + a1_ref[0]) / deg
    h = (jnp.dot(x_ref[...], w0a_ref[...], preferred_element_type=jnp.float32)
         + jnp.dot(agg, w0b_ref[...], preferred_element_type=jnp.float32)
         + b0_ref[...])
    h = jnp.maximum(h, 0.0)
    o = jnp.dot(h, w1_ref[...], preferred_element_type=jnp.float32) + b1_ref[...]
    n2 = jnp.sum(o * o, axis=1, keepdims=True)
    out_ref[...] = jnp.transpose(o * lax.rsqrt(jnp.maximum(n2, 1e-24)))


def _tc_mlp(x, agg_p, degt, W0, b0, W1, b1):
    grid = (-(-N_NODES // BN),)
    return pl.pallas_call(
        _tc_mlp_body,
        grid=grid,
        in_specs=[
            pl.BlockSpec((BN, D_FEAT), lambda i: (i, 0)),
            pl.BlockSpec((1, BN, D_FEAT), lambda i: (0, i, 0)),
            pl.BlockSpec((1, BN, D_FEAT), lambda i: (1, i, 0)),
            pl.BlockSpec((BN, 2), lambda i: (i, 0)),
            pl.BlockSpec((D_FEAT, HIDDEN), lambda i: (0, 0)),
            pl.BlockSpec((D_FEAT, HIDDEN), lambda i: (1, 0)),
            pl.BlockSpec((1, HIDDEN), lambda i: (0, 0)),
            pl.BlockSpec((HIDDEN, OUT_CH), lambda i: (0, 0)),
            pl.BlockSpec((1, OUT_CH), lambda i: (0, 0)),
        ],
        out_specs=pl.BlockSpec((OUT_CH, BN), lambda i: (0, i)),
        out_shape=jax.ShapeDtypeStruct((OUT_CH, N_NODES), jnp.float32),
    )(x, agg_p, agg_p, degt, W0, W0, b0, W1, b1)


def kernel(x, edge_index, W0, b0, W1, b1):
    ei3 = jnp.pad(edge_index.reshape(2, NW, EPW),
                  ((0, 0), (0, 0), (0, EPAD - EPW)))
    agg_p, deg_p = _sc_aggregate(x, ei3)
    degt = deg_p.reshape(NC, N_NODES).T
    out_t = _tc_mlp(x, agg_p, degt, W0, b0.reshape(1, HIDDEN),
                    W1, b1.reshape(1, OUT_CH))
    return out_t.T


# speedup vs baseline: 1.0158x; 1.0006x over previous
import functools

import jax
import jax.numpy as jnp
from jax import lax
from jax.experimental import pallas as pl
from jax.experimental.pallas import tpu as pltpu
from jax.experimental.pallas import tpu_sc as plsc

N_NODES = 10000
N_EDGES = 320000
D_FEAT = 128
HIDDEN = 256
OUT_CH = 64

NC = 2
NS = 16
NW = NC * NS
EPW = N_EDGES // NW
CH = 80
NCHUNK = EPW // CH
SPAN = 32
EPAD = 10112
RPT = 1000
ZROWS = 40
DEGC = 1000


def _sc_aggregate(x, ei3):
    mesh = plsc.VectorSubcoreMesh(core_axis_name="c", subcore_axis_name="s")

    @functools.partial(
        pl.kernel,
        out_type=[
            jax.ShapeDtypeStruct((NC, N_NODES, D_FEAT), jnp.float32),
            jax.ShapeDtypeStruct((NC * N_NODES,), jnp.float32),
        ],
        mesh=mesh,
        scratch_types=[
            pltpu.VMEM((SPAN * CH,), jnp.int32),
            pltpu.VMEM((SPAN * CH,), jnp.int32),
            pltpu.VMEM((CH, D_FEAT), jnp.float32),
            pltpu.VMEM((CH, D_FEAT), jnp.float32),
            pltpu.VMEM((CH, D_FEAT), jnp.float32),
            pltpu.VMEM((CH,), jnp.float32),
            pltpu.VMEM((1008,), jnp.float32),
            pltpu.VMEM_SHARED((N_NODES, D_FEAT), jnp.float32),
            pltpu.VMEM_SHARED((N_NODES,), jnp.float32),
            pltpu.SemaphoreType.DMA,
            pltpu.SemaphoreType.DMA,
            pltpu.SemaphoreType.DMA,
            pltpu.SemaphoreType.DMA,
            pltpu.SemaphoreType.DMA,
            pltpu.SemaphoreType.DMA,
            pltpu.SemaphoreType.DMA,
        ],
    )
    def k(x_hbm, ei_hbm, agg_out, deg_out,
          srcv, dstv, rows, rows1, rows2, ones, zdeg, agg_sh, deg_sh,
          gsem0, gsem1, gsem2, ssem0, ssem1, ssem2, osem):
        c = lax.axis_index("c")
        s = lax.axis_index("s")
        wid = c * NS + s

        zeros16 = jnp.zeros((16,), jnp.float32)

        def zero_rows(i, _):
            r = i // (D_FEAT // 16)
            col = (i % (D_FEAT // 16)) * 16
            rows[r, pl.ds(col, 16)] = zeros16
            return 0

        lax.fori_loop(0, CH * (D_FEAT // 16), zero_rows, 0)

        def zero_zdeg(i, _):
            zdeg[pl.ds(i * 16, 16)] = zeros16
            return 0

        lax.fori_loop(0, 1008 // 16, zero_zdeg, 0)

        ones16 = jnp.ones((16,), jnp.float32)

        def fill_ones(i, _):
            ones[pl.ds(i * 16, 16)] = ones16
            return 0

        lax.fori_loop(0, CH // 16, fill_ones, 0)

        @pl.when(s < N_NODES // RPT)
        def _():
            def zero_agg(t, _):
                pltpu.async_copy(rows.at[pl.ds(0, ZROWS), :],
                                 agg_sh.at[pl.ds(s * RPT + t * ZROWS, ZROWS), :],
                                 ssem0)
                return 0

            lax.fori_loop(0, RPT // ZROWS, zero_agg, 0)
            pltpu.sync_copy(zdeg.at[pl.ds(0, DEGC)], deg_sh.at[pl.ds(s * DEGC, DEGC)])

            def drain_zero(t, _):
                pltpu.make_async_copy(rows.at[pl.ds(0, ZROWS), :],
                                      agg_sh.at[pl.ds(s * RPT, ZROWS), :],
                                      ssem0).wait()
                return 0

            lax.fori_loop(0, RPT // ZROWS, drain_zero, 0)

        plsc.subcore_barrier()

        bufs = (rows, rows1, rows2)
        gsems = (gsem0, gsem1, gsem2)
        ssems = (ssem0, ssem1, ssem2)

        def run_span(base, nch):
            ne = -(-nch * CH // 128) * 128
            pltpu.sync_copy(ei_hbm.at[0, wid, pl.ds(base * CH, ne)],
                            srcv.at[pl.ds(0, ne)])
            pltpu.sync_copy(ei_hbm.at[1, wid, pl.ds(base * CH, ne)],
                            dstv.at[pl.ds(0, ne)])

            def sidx(c):
                return srcv.at[pl.ds(c * CH, CH)]

            def didx(c):
                return dstv.at[pl.ds(c * CH, CH)]

            pltpu.async_copy(x_hbm.at[sidx(0)], rows, gsem0)

            def triple(t, _):
                for j in range(3):
                    c = 3 * t + j
                    jn = (j + 1) % 3

                    @pl.when(jnp.logical_and(c >= 2, c < nch))
                    def _(jn=jn):
                        pltpu.make_async_copy(
                            bufs[jn], agg_sh.at[didx(0)], ssems[jn]).wait()

                    @pl.when(c + 1 < nch)
                    def _(jn=jn, c=c):
                        pltpu.async_copy(x_hbm.at[sidx(c + 1)], bufs[jn],
                                         gsems[jn])

                    @pl.when(c < nch)
                    def _(j=j, c=c):
                        pltpu.make_async_copy(x_hbm.at[sidx(0)], bufs[j],
                                              gsems[j]).wait()
                        pltpu.async_copy(ones, deg_sh.at[didx(c)], osem,
                                         add=True)
                        pltpu.async_copy(bufs[j], agg_sh.at[didx(c)],
                                         ssems[j], add=True)

                    @pl.when(jnp.logical_and(c >= 2, c < nch))
                    def _():
                        pltpu.make_async_copy(ones, deg_sh.at[didx(0)],
                                              osem).wait()
                return 0

            lax.fori_loop(0, (nch + 2) // 3, triple, 0)

            for c in (nch - 2, nch - 1):
                pltpu.make_async_copy(bufs[c % 3], agg_sh.at[didx(0)],
                                      ssems[c % 3]).wait()
                pltpu.make_async_copy(ones, deg_sh.at[didx(0)], osem).wait()

        run_span(0, SPAN)
        run_span(SPAN, SPAN)
        run_span(2 * SPAN, SPAN)
        run_span(3 * SPAN, NCHUNK - 3 * SPAN)

        plsc.subcore_barrier()

        @pl.when(s < N_NODES // RPT)
        def _():
            pltpu.async_copy(agg_sh.at[pl.ds(s * RPT, RPT), :],
                             agg_out.at[c, pl.ds(s * RPT, RPT), :], ssem0)
            pltpu.sync_copy(deg_sh.at[pl.ds(s * DEGC, DEGC)], zdeg.at[pl.ds(0, DEGC)])
            pltpu.sync_copy(zdeg.at[pl.ds(0, DEGC)],
                            deg_out.at[pl.ds(c * N_NODES + s * DEGC, DEGC)])
            pltpu.make_async_copy(agg_sh.at[pl.ds(s * RPT, RPT), :],
                                  agg_out.at[c, pl.ds(s * RPT, RPT), :],
                                  ssem0).wait()

    return k(x, ei3)


BN = 2560


def _tc_mlp_body(x_ref, a0_ref, a1_ref, degt_ref, w0a_ref, w0b_ref, b0_ref,
                 w1_ref, b1_ref, out_ref):
    deg = jnp.maximum(degt_ref[:, 0:1] + degt_ref[:, 1:2], 1.0)
    agg = (a0_ref[0] + a1_ref[0]) / deg
    h = (jnp.dot(x_ref[...], w0a_ref[...], preferred_element_type=jnp.float32)
         + jnp.dot(agg, w0b_ref[...], preferred_element_type=jnp.float32)
         + b0_ref[...])
    h = jnp.maximum(h, 0.0)
    o = jnp.dot(h, w1_ref[...], preferred_element_type=jnp.float32) + b1_ref[...]
    n2 = jnp.sum(o * o, axis=1, keepdims=True)
    out_ref[...] = jnp.transpose(o * lax.rsqrt(jnp.maximum(n2, 1e-24)))


def _tc_mlp(x, agg_p, degt, W0, b0, W1, b1):
    grid = (-(-N_NODES // BN),)
    return pl.pallas_call(
        _tc_mlp_body,
        grid=grid,
        in_specs=[
            pl.BlockSpec((BN, D_FEAT), lambda i: (i, 0)),
            pl.BlockSpec((1, BN, D_FEAT), lambda i: (0, i, 0)),
            pl.BlockSpec((1, BN, D_FEAT), lambda i: (1, i, 0)),
            pl.BlockSpec((BN, 2), lambda i: (i, 0)),
            pl.BlockSpec((D_FEAT, HIDDEN), lambda i: (0, 0)),
            pl.BlockSpec((D_FEAT, HIDDEN), lambda i: (1, 0)),
            pl.BlockSpec((1, HIDDEN), lambda i: (0, 0)),
            pl.BlockSpec((HIDDEN, OUT_CH), lambda i: (0, 0)),
            pl.BlockSpec((1, OUT_CH), lambda i: (0, 0)),
        ],
        out_specs=pl.BlockSpec((OUT_CH, BN), lambda i: (0, i)),
        out_shape=jax.ShapeDtypeStruct((OUT_CH, N_NODES), jnp.float32),
    )(x, agg_p, agg_p, degt, W0, W0, b0, W1, b1)


def kernel(x, edge_index, W0, b0, W1, b1):
    ei3 = jnp.pad(edge_index.reshape(2, NW, EPW),
                  ((0, 0), (0, 0), (0, EPAD - EPW)))
    agg_p, deg_p = _sc_aggregate(x, ei3)
    degt = deg_p.reshape(NC, N_NODES).T
    out_t = _tc_mlp(x, agg_p, degt, W0, b0.reshape(1, HIDDEN),
                    W1, b1.reshape(1, OUT_CH))
    return out_t.T
